# baseline (speedup 1.0000x reference)
import functools

import jax
import jax.numpy as jnp
from jax.experimental import pallas as pl
from jax.experimental.pallas import tpu as pltpu

_Q = 4
_BBG = 16
_NP = 88


def _tgcn_body(nt, x_ref, m_ref, alo_ref, ahi_ref, c3_ref, uzr_ref, uh_ref,
               lw_ref, lb_ref, out_ref, mx_ref):
    n, t, hid = nt
    npad = m_ref.shape[0]
    bbg = x_ref.shape[0]
    lpt = x_ref.shape[2] // t
    qh = _Q * hid
    m = m_ref[...]
    x = x_ref[...]
    for g in range(bbg):
        mx_ref[g * npad:(g + 1) * npad, :] = jnp.dot(
            m, x[g], preferred_element_type=jnp.float32
        ).astype(jnp.bfloat16)
    mx = mx_ref[...]
    alo = alo_ref[...]
    ahi = ahi_ref[...]
    c3 = c3_ref[...]
    uzr = uzr_ref[...]
    uh = uh_ref[...]
    rows = bbg * npad
    hs = jnp.zeros((rows, qh), dtype=jnp.float32)
    for ti in range(t):
        w = (ti // 2) * 2 * lpt
        a3 = alo if ti % 2 == 0 else ahi
        g3 = jnp.dot(mx[:, w:w + 2 * lpt], a3,
                     preferred_element_type=jnp.float32) + c3
        zr = g3[:, :2 * qh] + jnp.dot(hs.astype(jnp.bfloat16), uzr,
                                      preferred_element_type=jnp.float32)
        z = 0.5 + 0.5 * jnp.tanh(0.5 * zr[:, :qh])
        r = 0.5 + 0.5 * jnp.tanh(0.5 * zr[:, qh:2 * qh])
        ht = jnp.tanh(g3[:, 2 * qh:] + jnp.dot(
            (hs * r).astype(jnp.bfloat16), uh,
            preferred_element_type=jnp.float32))
        hs = z * hs + (1.0 - z) * ht
    hr = jax.nn.relu(hs).reshape(bbg, npad, qh)
    mask = (jax.lax.broadcasted_iota(jnp.int32, (bbg, npad, qh), 1)
            < n).astype(jnp.float32)
    pooled = jnp.sum(hr * mask, axis=1) * (1.0 / n)
    out_ref[...] = (jnp.dot(pooled, lw_ref[...],
                            preferred_element_type=jnp.float32)
                    + lb_ref[...])[None]


def _blockdiag(w, q):
    k, c = w.shape
    out = jnp.zeros((q * k, q * c), w.dtype)
    for b in range(q):
        out = out.at[b * k:(b + 1) * k, b * c:(b + 1) * c].set(w)
    return out


def kernel(agent_obs, hideout_obs, timestep_obs, num_agents,
           last_k_fugitive_detections, edge_index,
           Wz, bz, Wr, br, Wh, bh,
           Lzw, Lzb, Lrw, Lrb, Lhw, Lhb, lin_w, lin_b):
    B, T, N, F = agent_obs.shape
    HID = Wz.shape[1]
    PER = lin_w.shape[1]
    Q = _Q

    loop = jnp.arange(N)
    row = jnp.concatenate([edge_index[0].astype(jnp.int32), loop])
    col = jnp.concatenate([edge_index[1].astype(jnp.int32), loop])
    deg = jnp.zeros((N,), jnp.float32).at[col].add(1.0)
    dinv = jnp.where(deg > 0, 1.0 / jnp.sqrt(deg), 0.0)
    norm = dinv[row] * dinv[col]
    m = jnp.zeros((N, N), jnp.float32).at[col, row].add(norm)
    mp = jnp.zeros((_NP, N), jnp.float32).at[:N].set(m)
    mp = mp.astype(jnp.bfloat16)

    a3 = jnp.concatenate(
        [_blockdiag(Wz @ Lzw[:HID], Q), _blockdiag(Wr @ Lrw[:HID], Q),
         _blockdiag(Wh @ Lhw[:HID], Q)], axis=1)
    zqf = jnp.zeros((Q * F, 3 * Q * HID), jnp.float32)
    alo = jnp.concatenate([a3, zqf], axis=0).astype(jnp.bfloat16)
    ahi = jnp.concatenate([zqf, a3], axis=0).astype(jnp.bfloat16)
    c3 = jnp.concatenate(
        [jnp.tile(bz @ Lzw[:HID] + Lzb, Q), jnp.tile(br @ Lrw[:HID] + Lrb, Q),
         jnp.tile(bh @ Lhw[:HID] + Lhb, Q)])[None]
    uzr = jnp.concatenate(
        [_blockdiag(Lzw[HID:], Q), _blockdiag(Lrw[HID:], Q)],
        axis=1).astype(jnp.bfloat16)
    uh = _blockdiag(Lhw[HID:], Q).astype(jnp.bfloat16)
    lwbd = _blockdiag(lin_w, Q)
    lbbd = jnp.tile(lin_b, Q)[None]

    xq = agent_obs.reshape(B // Q, Q, T, N, F).transpose(0, 3, 2, 1, 4)
    xp = xq.reshape(B // Q, N, T * Q * F).astype(jnp.bfloat16)

    grid = ((B // Q) // _BBG,)
    pooled = pl.pallas_call(
        functools.partial(_tgcn_body, (N, T, HID)),
        grid=grid,
        in_specs=[
            pl.BlockSpec((_BBG, N, T * Q * F), lambda i: (i, 0, 0)),
            pl.BlockSpec((_NP, N), lambda i: (0, 0)),
            pl.BlockSpec((2 * Q * F, 3 * Q * HID), lambda i: (0, 0)),
            pl.BlockSpec((2 * Q * F, 3 * Q * HID), lambda i: (0, 0)),
            pl.BlockSpec((1, 3 * Q * HID), lambda i: (0, 0)),
            pl.BlockSpec((Q * HID, 2 * Q * HID), lambda i: (0, 0)),
            pl.BlockSpec((Q * HID, Q * HID), lambda i: (0, 0)),
            pl.BlockSpec((Q * HID, Q * PER), lambda i: (0, 0)),
            pl.BlockSpec((1, Q * PER), lambda i: (0, 0)),
        ],
        out_specs=pl.BlockSpec((1, _BBG, Q * PER), lambda i: (i, 0, 0)),
        out_shape=jax.ShapeDtypeStruct((B // Q // _BBG, _BBG, Q * PER),
                                       jnp.float32),
        scratch_shapes=[pltpu.VMEM((_BBG * _NP, T * Q * F), jnp.bfloat16)],
        compiler_params=pltpu.CompilerParams(
            dimension_semantics=("parallel",)),
    )(xp, mp, alo, ahi, c3, uzr, uh, lwbd, lbbd)

    return jnp.concatenate(
        [pooled.reshape(B, PER), hideout_obs, timestep_obs,
         last_k_fugitive_detections], axis=-1)


# speedup vs baseline: 1.4698x; 1.4698x over previous
import functools

import jax
import jax.numpy as jnp
from jax.experimental import pallas as pl
from jax.experimental.pallas import tpu as pltpu

_Q = 4
_BBG = 16
_NP = 88


def _tgcn_body(nt, x_ref, m_ref, alo_ref, ahi_ref, c3_ref, uzr_ref, uh_ref,
               lw_ref, lb_ref, out_ref):
    n, t, hid = nt
    npad = x_ref.shape[1]
    bbg = x_ref.shape[0]
    lpt = x_ref.shape[2] // t
    qh = _Q * hid
    m = m_ref[...]
    x = x_ref[...]
    mx = jnp.concatenate(
        [jnp.dot(m, x[g], preferred_element_type=jnp.float32)
         for g in range(bbg)], axis=0).astype(jnp.bfloat16)
    alo = alo_ref[...]
    ahi = ahi_ref[...]
    c3 = c3_ref[...]
    uzr = uzr_ref[...]
    uh = uh_ref[...]
    rows = bbg * npad
    hs = jnp.zeros((rows, qh), dtype=jnp.float32)
    for ti in range(t):
        w = (ti // 2) * 2 * lpt
        a3 = alo if ti % 2 == 0 else ahi
        g3 = jnp.dot(mx[:, w:w + 2 * lpt], a3,
                     preferred_element_type=jnp.float32) + c3
        zr = g3[:, :2 * qh] + jnp.dot(hs.astype(jnp.bfloat16), uzr,
                                      preferred_element_type=jnp.float32)
        z = 0.5 + 0.5 * jnp.tanh(0.5 * zr[:, :qh])
        r = 0.5 + 0.5 * jnp.tanh(0.5 * zr[:, qh:2 * qh])
        ht = jnp.tanh(g3[:, 2 * qh:] + jnp.dot(
            (hs * r).astype(jnp.bfloat16), uh,
            preferred_element_type=jnp.float32))
        hs = z * hs + (1.0 - z) * ht
    hr = jax.nn.relu(hs).reshape(bbg, npad, qh)
    mask = (jax.lax.broadcasted_iota(jnp.int32, (bbg, npad, qh), 1)
            < n).astype(jnp.float32)
    pooled = jnp.sum(hr * mask, axis=1) * (1.0 / n)
    out_ref[...] = (jnp.dot(pooled, lw_ref[...],
                            preferred_element_type=jnp.float32)
                    + lb_ref[...])[None]


def _blockdiag(w, q):
    k, c = w.shape
    return (jnp.eye(q, dtype=w.dtype)[:, None, :, None]
            * w[None, :, None, :]).reshape(q * k, q * c)


def kernel(agent_obs, hideout_obs, timestep_obs, num_agents,
           last_k_fugitive_detections, edge_index,
           Wz, bz, Wr, br, Wh, bh,
           Lzw, Lzb, Lrw, Lrb, Lhw, Lhb, lin_w, lin_b):
    B, T, N, F = agent_obs.shape
    HID = Wz.shape[1]
    PER = lin_w.shape[1]
    Q = _Q

    loop = jnp.arange(N)
    row = jnp.concatenate([edge_index[0].astype(jnp.int32), loop])
    col = jnp.concatenate([edge_index[1].astype(jnp.int32), loop])
    ids = jnp.arange(N, dtype=jnp.int32)
    oh_row = (row[:, None] == ids[None, :]).astype(jnp.float32)
    oh_col = (col[:, None] == ids[None, :]).astype(jnp.float32)
    deg = jnp.sum(oh_col, axis=0)
    dinv = jnp.where(deg > 0, 1.0 / jnp.sqrt(deg), 0.0)
    norm = (oh_row @ dinv) * (oh_col @ dinv)
    m = oh_col.T @ (oh_row * norm[:, None])
    mp = jnp.pad(m, ((0, _NP - N), (0, _NP - N))).astype(jnp.bfloat16)

    a3 = jnp.concatenate(
        [_blockdiag(Wz @ Lzw[:HID], Q), _blockdiag(Wr @ Lrw[:HID], Q),
         _blockdiag(Wh @ Lhw[:HID], Q)], axis=1)
    zqf = jnp.zeros((Q * F, 3 * Q * HID), jnp.float32)
    alo = jnp.concatenate([a3, zqf], axis=0).astype(jnp.bfloat16)
    ahi = jnp.concatenate([zqf, a3], axis=0).astype(jnp.bfloat16)
    c3 = jnp.concatenate(
        [jnp.tile(bz @ Lzw[:HID] + Lzb, Q), jnp.tile(br @ Lrw[:HID] + Lrb, Q),
         jnp.tile(bh @ Lhw[:HID] + Lhb, Q)])[None]
    uzr = jnp.concatenate(
        [_blockdiag(Lzw[HID:], Q), _blockdiag(Lrw[HID:], Q)],
        axis=1).astype(jnp.bfloat16)
    uh = _blockdiag(Lhw[HID:], Q).astype(jnp.bfloat16)
    lwbd = _blockdiag(lin_w, Q)
    lbbd = jnp.tile(lin_b, Q)[None]

    xq = agent_obs.reshape(B // Q, Q, T, N, F).transpose(0, 3, 2, 1, 4)
    xq = xq.reshape(B // Q, N, T * Q * F).astype(jnp.bfloat16)
    xp = jax.lax.pad(xq, jnp.bfloat16(0),
                     ((0, 0, 0), (0, _NP - N, 0), (0, 0, 0)))

    grid = ((B // Q) // _BBG,)
    pooled = pl.pallas_call(
        functools.partial(_tgcn_body, (N, T, HID)),
        grid=grid,
        in_specs=[
            pl.BlockSpec((_BBG, _NP, T * Q * F), lambda i: (i, 0, 0)),
            pl.BlockSpec((_NP, _NP), lambda i: (0, 0)),
            pl.BlockSpec((2 * Q * F, 3 * Q * HID), lambda i: (0, 0)),
            pl.BlockSpec((2 * Q * F, 3 * Q * HID), lambda i: (0, 0)),
            pl.BlockSpec((1, 3 * Q * HID), lambda i: (0, 0)),
            pl.BlockSpec((Q * HID, 2 * Q * HID), lambda i: (0, 0)),
            pl.BlockSpec((Q * HID, Q * HID), lambda i: (0, 0)),
            pl.BlockSpec((Q * HID, Q * PER), lambda i: (0, 0)),
            pl.BlockSpec((1, Q * PER), lambda i: (0, 0)),
        ],
        out_specs=pl.BlockSpec((1, _BBG, Q * PER), lambda i: (i, 0, 0)),
        out_shape=jax.ShapeDtypeStruct((B // Q // _BBG, _BBG, Q * PER),
                                       jnp.float32),
        compiler_params=pltpu.CompilerParams(
            dimension_semantics=("parallel",)),
    )(xp, mp, alo, ahi, c3, uzr, uh, lwbd, lbbd)

    return jnp.concatenate(
        [pooled.reshape(B, PER), hideout_obs, timestep_obs,
         last_k_fugitive_detections], axis=-1)
